# baseline (speedup 1.0000x reference)
import jax
import jax.numpy as jnp
from jax.experimental import pallas as pl
from jax.experimental.pallas import tpu as pltpu

_K = 8
_SB = 512
_CB = 128
_P = 4


def _topk_body(x_ref, o_ref, t_ref):
    s = pl.program_id(2)
    ns = pl.num_programs(2)

    neg = jnp.float32(-jnp.inf)
    init = jnp.full((_P * _K, 8, _CB), neg, jnp.float32)
    T = jnp.where(s == 0, init, t_ref[...])
    Ts = [[T[p * _K + j] for j in range(_K)] for p in range(_P)]

    x = x_ref[0]
    for i in range(_SB // (8 * _P)):
        for p in range(_P):
            g = i * _P + p
            v = x[g * 8:(g + 1) * 8, :]
            for j in range(_K):
                hi = jnp.maximum(Ts[p][j], v)
                if j < _K - 1:
                    v = jnp.minimum(Ts[p][j], v)
                Ts[p][j] = hi
    t_ref[...] = jnp.stack([t for row in Ts for t in row])

    @pl.when(s == ns - 1)
    def _():
        nc = _P * _K * 8
        cand = t_ref[...].reshape(nc, _CB)
        rows = jax.lax.broadcasted_iota(jnp.int32, (nc, _CB), 0)
        outs = []
        c = cand
        for j in range(_K):
            m = jnp.max(c, axis=0, keepdims=True)
            outs.append(m)
            if j < _K - 1:
                eq = c == m
                idx = jnp.where(eq, rows, nc)
                amin = jnp.min(idx, axis=0, keepdims=True)
                c = jnp.where(rows == amin, neg, c)
        o_ref[0] = jnp.concatenate(outs, axis=0)


def kernel(inputs):
    B, S, C = inputs.shape
    grid = (B, C // _CB, S // _SB)
    out3 = pl.pallas_call(
        _topk_body,
        grid=grid,
        in_specs=[pl.BlockSpec((1, _SB, _CB), lambda b, c, s: (b, s, c))],
        out_specs=pl.BlockSpec((1, _K, _CB), lambda b, c, s: (b, 0, c)),
        out_shape=jax.ShapeDtypeStruct((B, _K, C), jnp.float32),
        scratch_shapes=[pltpu.VMEM((_P * _K, 8, _CB), jnp.float32)],
        compiler_params=pltpu.CompilerParams(
            dimension_semantics=("parallel", "parallel", "arbitrary")),
    )(inputs)
    return jnp.transpose(out3, (0, 2, 1)).reshape(B, C * _K)


# speedup vs baseline: 3.7872x; 3.7872x over previous
import jax
import jax.numpy as jnp
from jax.experimental import pallas as pl
from jax.experimental.pallas import tpu as pltpu

_K = 8
_CB = 128
_P = 4
_NC = _P * _K * 8


def _topk_body(x_ref, o_ref):
    S = x_ref.shape[1]
    neg = jnp.float32(-jnp.inf)
    Ts = [[jnp.full((8, _CB), neg, jnp.float32) for _ in range(_K)]
          for _ in range(_P)]

    x = x_ref[0]
    for i in range(S // (8 * _P)):
        for p in range(_P):
            g = i * _P + p
            v = x[g * 8:(g + 1) * 8, :]
            for j in range(_K):
                hi = jnp.maximum(Ts[p][j], v)
                if j < _K - 1:
                    v = jnp.minimum(Ts[p][j], v)
                Ts[p][j] = hi

    cand = jnp.concatenate([t for row in Ts for t in row], axis=0)
    rows = jax.lax.broadcasted_iota(jnp.int32, (_NC, _CB), 0)
    outs = []
    c = cand
    for j in range(_K):
        m = jnp.max(c, axis=0, keepdims=True)
        outs.append(m)
        if j < _K - 1:
            eq = c == m
            idx = jnp.where(eq, rows, _NC)
            amin = jnp.min(idx, axis=0, keepdims=True)
            c = jnp.where(rows == amin, neg, c)
    o_ref[0] = jnp.concatenate(outs, axis=0)


def kernel(inputs):
    B, S, C = inputs.shape
    out3 = pl.pallas_call(
        _topk_body,
        grid=(B, C // _CB),
        in_specs=[pl.BlockSpec((1, S, _CB), lambda b, c: (b, 0, c))],
        out_specs=pl.BlockSpec((1, _K, _CB), lambda b, c: (b, 0, c)),
        out_shape=jax.ShapeDtypeStruct((B, _K, C), jnp.float32),
        compiler_params=pltpu.CompilerParams(
            dimension_semantics=("parallel", "parallel")),
    )(inputs)
    return jnp.transpose(out3, (0, 2, 1)).reshape(B, C * _K)


# speedup vs baseline: 4.5541x; 1.2025x over previous
import jax
import jax.numpy as jnp
from jax.experimental import pallas as pl
from jax.experimental.pallas import tpu as pltpu

_K = 8
_CB = 128
_P = 2
_DEPTHS = (8, 4, 2, 2, 1, 1, 1, 1)
_NC = _P * sum(_DEPTHS) * 8

_SORT8 = ((0, 1), (2, 3), (4, 5), (6, 7),
          (0, 2), (1, 3), (4, 6), (5, 7),
          (1, 2), (5, 6),
          (0, 4), (1, 5), (2, 6), (3, 7),
          (2, 4), (3, 5),
          (1, 2), (3, 4), (5, 6))


def _topk_body(x_ref, o_ref):
    S = x_ref.shape[1]
    neg = jnp.float32(-jnp.inf)
    Ts = [[[jnp.full((8, _CB), neg, jnp.float32) for _ in range(d)]
           for d in _DEPTHS] for _ in range(_P)]

    x = x_ref[0]
    for i in range(S // (64 * _P)):
        for p in range(_P):
            base = (i * _P + p) * 64
            vs = [x[base + q * 8: base + (q + 1) * 8, :] for q in range(8)]
            for a, b in _SORT8:
                hi = jnp.maximum(vs[a], vs[b])
                vs[b] = jnp.minimum(vs[a], vs[b])
                vs[a] = hi
            for r in range(8):
                T = Ts[p][r]
                v = vs[r]
                for j in range(_DEPTHS[r]):
                    hi = jnp.maximum(T[j], v)
                    if j < _DEPTHS[r] - 1:
                        v = jnp.minimum(T[j], v)
                    T[j] = hi

    cand = jnp.concatenate(
        [t for ts in Ts for chain in ts for t in chain], axis=0)
    rows = jax.lax.broadcasted_iota(jnp.int32, (_NC, _CB), 0)
    outs = []
    c = cand
    for j in range(_K):
        m = jnp.max(c, axis=0, keepdims=True)
        outs.append(m)
        if j < _K - 1:
            eq = c == m
            idx = jnp.where(eq, rows, _NC)
            amin = jnp.min(idx, axis=0, keepdims=True)
            c = jnp.where(rows == amin, neg, c)
    o_ref[0] = jnp.concatenate(outs, axis=0)


def kernel(inputs):
    B, S, C = inputs.shape
    out3 = pl.pallas_call(
        _topk_body,
        grid=(B, C // _CB),
        in_specs=[pl.BlockSpec((1, S, _CB), lambda b, c: (b, 0, c))],
        out_specs=pl.BlockSpec((1, _K, _CB), lambda b, c: (b, 0, c)),
        out_shape=jax.ShapeDtypeStruct((B, _K, C), jnp.float32),
        compiler_params=pltpu.CompilerParams(
            dimension_semantics=("parallel", "parallel")),
    )(inputs)
    return jnp.transpose(out3, (0, 2, 1)).reshape(B, C * _K)


# speedup vs baseline: 4.7596x; 1.0451x over previous
import jax
import jax.numpy as jnp
from jax.experimental import pallas as pl
from jax.experimental.pallas import tpu as pltpu

_K = 8
_CB = 128
_P = 1
_DEPTHS = (8, 4, 2, 2, 1, 1, 1, 1)
_NC = _P * sum(_DEPTHS) * 8

_SORT8 = ((0, 1), (2, 3), (4, 5), (6, 7),
          (0, 2), (1, 3), (4, 6), (5, 7),
          (1, 2), (5, 6),
          (0, 4), (1, 5), (2, 6), (3, 7),
          (2, 4), (3, 5),
          (1, 2), (3, 4), (5, 6))


def _topk_body(x_ref, o_ref):
    S = x_ref.shape[1]
    neg = jnp.float32(-jnp.inf)
    Ts = [[[jnp.full((8, _CB), neg, jnp.float32) for _ in range(d)]
           for d in _DEPTHS] for _ in range(_P)]

    x = x_ref[0]
    for i in range(S // (64 * _P)):
        for p in range(_P):
            base = (i * _P + p) * 64
            vs = [x[base + q * 8: base + (q + 1) * 8, :] for q in range(8)]
            for a, b in _SORT8:
                hi = jnp.maximum(vs[a], vs[b])
                vs[b] = jnp.minimum(vs[a], vs[b])
                vs[a] = hi
            for r in range(8):
                T = Ts[p][r]
                v = vs[r]
                for j in range(_DEPTHS[r]):
                    hi = jnp.maximum(T[j], v)
                    if j < _DEPTHS[r] - 1:
                        v = jnp.minimum(T[j], v)
                    T[j] = hi

    cand = jnp.concatenate(
        [t for ts in Ts for chain in ts for t in chain], axis=0)
    rows = jax.lax.broadcasted_iota(jnp.int32, (_NC, _CB), 0)
    outs = []
    c = cand
    for j in range(_K):
        m = jnp.max(c, axis=0, keepdims=True)
        outs.append(m)
        if j < _K - 1:
            eq = c == m
            idx = jnp.where(eq, rows, _NC)
            amin = jnp.min(idx, axis=0, keepdims=True)
            c = jnp.where(rows == amin, neg, c)
    o_ref[0] = jnp.concatenate(outs, axis=0)


def kernel(inputs):
    B, S, C = inputs.shape
    out3 = pl.pallas_call(
        _topk_body,
        grid=(B, C // _CB),
        in_specs=[pl.BlockSpec((1, S, _CB), lambda b, c: (b, 0, c))],
        out_specs=pl.BlockSpec((1, _K, _CB), lambda b, c: (b, 0, c)),
        out_shape=jax.ShapeDtypeStruct((B, _K, C), jnp.float32),
        compiler_params=pltpu.CompilerParams(
            dimension_semantics=("parallel", "parallel")),
    )(inputs)
    return jnp.transpose(out3, (0, 2, 1)).reshape(B, C * _K)


# speedup vs baseline: 5.4012x; 1.1348x over previous
import jax
import jax.numpy as jnp
from jax.experimental import pallas as pl
from jax.experimental.pallas import tpu as pltpu

_K = 8
_CB = 256
_P = 1
_DEPTHS = (8, 4, 2, 2, 1, 1, 1, 1)
_NC = _P * sum(_DEPTHS) * 8

_SORT8 = ((0, 1), (2, 3), (4, 5), (6, 7),
          (0, 2), (1, 3), (4, 6), (5, 7),
          (1, 2), (5, 6),
          (0, 4), (1, 5), (2, 6), (3, 7),
          (2, 4), (3, 5),
          (1, 2), (3, 4), (5, 6))


def _topk_body(x_ref, o_ref):
    S = x_ref.shape[1]
    neg = jnp.float32(-jnp.inf)
    Ts = [[[jnp.full((8, _CB), neg, jnp.float32) for _ in range(d)]
           for d in _DEPTHS] for _ in range(_P)]

    x = x_ref[0]
    for i in range(S // (64 * _P)):
        for p in range(_P):
            base = (i * _P + p) * 64
            vs = [x[base + q * 8: base + (q + 1) * 8, :] for q in range(8)]
            for a, b in _SORT8:
                hi = jnp.maximum(vs[a], vs[b])
                vs[b] = jnp.minimum(vs[a], vs[b])
                vs[a] = hi
            for r in range(8):
                T = Ts[p][r]
                v = vs[r]
                for j in range(_DEPTHS[r]):
                    hi = jnp.maximum(T[j], v)
                    if j < _DEPTHS[r] - 1:
                        v = jnp.minimum(T[j], v)
                    T[j] = hi

    cand = jnp.concatenate(
        [t for ts in Ts for chain in ts for t in chain], axis=0)
    rows = jax.lax.broadcasted_iota(jnp.int32, (_NC, _CB), 0)
    outs = []
    c = cand
    for j in range(_K):
        m = jnp.max(c, axis=0, keepdims=True)
        outs.append(m)
        if j < _K - 1:
            eq = c == m
            idx = jnp.where(eq, rows, _NC)
            amin = jnp.min(idx, axis=0, keepdims=True)
            c = jnp.where(rows == amin, neg, c)
    o_ref[0] = jnp.concatenate(outs, axis=0)


def kernel(inputs):
    B, S, C = inputs.shape
    out3 = pl.pallas_call(
        _topk_body,
        grid=(B, C // _CB),
        in_specs=[pl.BlockSpec((1, S, _CB), lambda b, c: (b, 0, c))],
        out_specs=pl.BlockSpec((1, _K, _CB), lambda b, c: (b, 0, c)),
        out_shape=jax.ShapeDtypeStruct((B, _K, C), jnp.float32),
        compiler_params=pltpu.CompilerParams(
            dimension_semantics=("parallel", "parallel")),
    )(inputs)
    return jnp.transpose(out3, (0, 2, 1)).reshape(B, C * _K)


# speedup vs baseline: 5.5530x; 1.0281x over previous
import jax
import jax.numpy as jnp
from jax.experimental import pallas as pl
from jax.experimental.pallas import tpu as pltpu

_K = 8
_CB = 512
_P = 1
_DEPTHS = (8, 4, 2, 2, 1, 1, 1, 1)
_NC = _P * sum(_DEPTHS) * 8

_SORT8 = ((0, 1), (2, 3), (4, 5), (6, 7),
          (0, 2), (1, 3), (4, 6), (5, 7),
          (1, 2), (5, 6),
          (0, 4), (1, 5), (2, 6), (3, 7),
          (2, 4), (3, 5),
          (1, 2), (3, 4), (5, 6))


def _topk_body(x_ref, o_ref):
    S = x_ref.shape[1]
    neg = jnp.float32(-jnp.inf)
    Ts = [[[jnp.full((8, _CB), neg, jnp.float32) for _ in range(d)]
           for d in _DEPTHS] for _ in range(_P)]

    x = x_ref[0]
    for i in range(S // (64 * _P)):
        for p in range(_P):
            base = (i * _P + p) * 64
            vs = [x[base + q * 8: base + (q + 1) * 8, :] for q in range(8)]
            for a, b in _SORT8:
                hi = jnp.maximum(vs[a], vs[b])
                vs[b] = jnp.minimum(vs[a], vs[b])
                vs[a] = hi
            for r in range(8):
                T = Ts[p][r]
                v = vs[r]
                for j in range(_DEPTHS[r]):
                    hi = jnp.maximum(T[j], v)
                    if j < _DEPTHS[r] - 1:
                        v = jnp.minimum(T[j], v)
                    T[j] = hi

    cand = jnp.concatenate(
        [t for ts in Ts for chain in ts for t in chain], axis=0)
    rows = jax.lax.broadcasted_iota(jnp.int32, (_NC, _CB), 0)
    outs = []
    c = cand
    for j in range(_K):
        m = jnp.max(c, axis=0, keepdims=True)
        outs.append(m)
        if j < _K - 1:
            eq = c == m
            idx = jnp.where(eq, rows, _NC)
            amin = jnp.min(idx, axis=0, keepdims=True)
            c = jnp.where(rows == amin, neg, c)
    o_ref[0] = jnp.concatenate(outs, axis=0)


def kernel(inputs):
    B, S, C = inputs.shape
    out3 = pl.pallas_call(
        _topk_body,
        grid=(B, C // _CB),
        in_specs=[pl.BlockSpec((1, S, _CB), lambda b, c: (b, 0, c))],
        out_specs=pl.BlockSpec((1, _K, _CB), lambda b, c: (b, 0, c)),
        out_shape=jax.ShapeDtypeStruct((B, _K, C), jnp.float32),
        compiler_params=pltpu.CompilerParams(
            dimension_semantics=("parallel", "parallel")),
    )(inputs)
    return jnp.transpose(out3, (0, 2, 1)).reshape(B, C * _K)
